# Initial kernel scaffold; baseline (speedup 1.0000x reference)
import jax
import jax.numpy as jnp
from jax.experimental import pallas as pl


def kernel(x, edge_index, batch, Ws, bs, gammas, betas, fcW, fcb):
    raise NotImplementedError("write your pallas kernel here")


# speedup vs baseline: 1.5814x; 1.5814x over previous
import functools

import jax
import jax.numpy as jnp
from jax import lax
from jax.experimental import pallas as pl
from jax.experimental.pallas import tpu as pltpu
from jax.experimental.pallas import tpu_sc as plsc

NSC = 2
NT = 16
NW = NSC * NT
EB = 128
NGRAPH = 16
EPS = 1e-5
NT_ROWS = 1000
CB = 1024


def _padc(c):
    return ((c + 127) // 128) * 128


def _pad_to(a, shape):
    return jnp.pad(a, [(0, t - s) for s, t in zip(a.shape, shape)])


def _sc_gather_msg(hp_flat, srcb_k, e_pad2, nb):
    mesh = plsc.VectorSubcoreMesh(core_axis_name="c", subcore_axis_name="s")
    per_w = nb * EB
    nz = (e_pad2 - NW * per_w - CB) // NW
    e_pad = NW * per_w

    @functools.partial(
        pl.kernel, mesh=mesh,
        out_type=jax.ShapeDtypeStruct((e_pad2, 128), jnp.float32),
        scratch_types=[
            pltpu.VMEM((nb, EB), jnp.int32),
            pltpu.VMEM((EB, 128), jnp.float32),
            pltpu.VMEM((max(nz, 8), 128), jnp.float32),
            pltpu.SemaphoreType.DMA,
        ],
    )
    def k(hp_hbm, src_hbm, out_hbm, idx, buf, zbuf, sem):
        c = lax.axis_index("c")
        s = lax.axis_index("s")
        w = s * NSC + c

        def zb(t, _):
            r = t // 8
            col = (t % 8) * 16
            zbuf[r, pl.ds(col, 16)] = jnp.zeros((16,), jnp.float32)
            return 0

        lax.fori_loop(0, max(nz, 8) * 8, zb, 0)
        pltpu.sync_copy(src_hbm.at[w], idx)

        def step(j, _):
            pltpu.async_copy(hp_hbm.at[idx.at[j]], buf, sem).wait()
            pltpu.sync_copy(buf, out_hbm.at[pl.ds(w * per_w + j * EB, EB)])
            return 0

        lax.fori_loop(0, nb, step, 0)
        if nz > 0:
            pltpu.sync_copy(zbuf.at[pl.ds(0, nz)],
                            out_hbm.at[pl.ds(e_pad + w * nz, nz)])

    return k(hp_flat, srcb_k)


def _sc_gather_bounds(p_tab, bidx, nbb):
    mesh = plsc.VectorSubcoreMesh(core_axis_name="c", subcore_axis_name="s")
    per_w = nbb * EB

    @functools.partial(
        pl.kernel, mesh=mesh,
        out_type=jax.ShapeDtypeStruct((2, NW * per_w, 128), jnp.float32),
        scratch_types=[
            pltpu.VMEM((nbb, EB), jnp.int32),
            pltpu.VMEM((EB, 128), jnp.float32),
            pltpu.SemaphoreType.DMA,
        ],
    )
    def k(p_hbm, bidx_hbm, out_hbm, idx, buf, sem):
        c = lax.axis_index("c")
        s = lax.axis_index("s")
        w = s * NSC + c
        for h in range(2):
            pltpu.sync_copy(bidx_hbm.at[h, w], idx)
            for j in range(nbb):
                pltpu.async_copy(p_hbm.at[idx.at[j]], buf, sem).wait()
                pltpu.sync_copy(
                    buf, out_hbm.at[h, pl.ds(w * per_w + j * EB, EB)])

    return k(p_tab, bidx)


def _tc_cumsum(msg, e_pad2):
    nblk = e_pad2 // CB

    def body(m_ref, p_ref, carry):
        i = pl.program_id(0)

        @pl.when(i == 0)
        def _():
            carry[...] = jnp.zeros((1, 128), jnp.float32)

        @pl.when(i < nblk)
        def _():
            x = m_ref[...]
            sh = 1
            while sh < CB:
                x = x + jnp.concatenate(
                    [jnp.zeros((sh, 128), jnp.float32), x[:CB - sh]], axis=0)
                sh *= 2
            y = x + carry[...]
            p_ref[...] = y
            carry[...] = y[CB - 1:CB, :]

        @pl.when(i == nblk)
        def _():
            p_ref[...] = jnp.zeros((CB, 128), jnp.float32)

    return pl.pallas_call(
        body,
        grid=(nblk + 1,),
        in_specs=[pl.BlockSpec((CB, 128),
                               lambda i: (jnp.minimum(i, nblk - 1), 0))],
        out_specs=pl.BlockSpec((CB, 128), lambda i: (i, 0)),
        out_shape=jax.ShapeDtypeStruct((e_pad2 + CB, 128), jnp.float32),
        scratch_shapes=[pltpu.VMEM((1, 128), jnp.float32)],
    )(msg)


def _tc_dis(olo, ohi, n):
    nbk = n // NT_ROWS

    def body(lo_ref, hi_ref, o_ref):
        deg = (hi_ref[...] - lo_ref[...]).astype(jnp.float32) + 1.0
        o_ref[...] = lax.rsqrt(deg)

    return pl.pallas_call(
        body,
        grid=(nbk,),
        in_specs=[pl.BlockSpec((NT_ROWS, 1), lambda i: (i, 0)),
                  pl.BlockSpec((NT_ROWS, 1), lambda i: (i, 0))],
        out_specs=pl.BlockSpec((NT_ROWS, 1), lambda i: (i, 0)),
        out_shape=jax.ShapeDtypeStruct((n, 1), jnp.float32),
    )(olo, ohi)


def _tc_matmul(a_ch, wmat, sums_in, gam_in, bet_in, dis, n, nc_in, nc_out):
    nbk = n // NT_ROWS
    c_in = nc_in * 128

    def body(a_ref, w_ref, s_ref, g_ref, be_ref, d_ref, o_ref):
        acc = None
        for ki in range(nc_in):
            s0 = s_ref[ki, 0, :]
            s1 = s_ref[ki, 1, :]
            mu = s0 / n
            var = s1 / n - mu * mu
            scale = g_ref[ki, 0, :] * lax.rsqrt(var + EPS)
            shift = be_ref[ki, 0, :] - mu * scale
            aeff = a_ref[ki] * scale[None, :] + shift[None, :]
            part = lax.dot_general(
                aeff, w_ref[ki * 128:(ki + 1) * 128, :],
                (((1,), (0,)), ((), ())), preferred_element_type=jnp.float32)
            acc = part if acc is None else acc + part
        o_ref[0] = acc * d_ref[...]

    return pl.pallas_call(
        body,
        grid=(nbk, nc_out),
        in_specs=[
            pl.BlockSpec((nc_in, NT_ROWS, 128), lambda i, k: (0, i, 0)),
            pl.BlockSpec((c_in, 128), lambda i, k: (0, k)),
            pl.BlockSpec((nc_in, 2, 128), lambda i, k: (0, 0, 0)),
            pl.BlockSpec((nc_in, 1, 128), lambda i, k: (0, 0, 0)),
            pl.BlockSpec((nc_in, 1, 128), lambda i, k: (0, 0, 0)),
            pl.BlockSpec((NT_ROWS, 1), lambda i, k: (i, 0)),
        ],
        out_specs=pl.BlockSpec((1, NT_ROWS, 128), lambda i, k: (k, i, 0)),
        out_shape=jax.ShapeDtypeStruct((nc_out, n, 128), jnp.float32),
    )(a_ch, wmat, sums_in, gam_in, bet_in, dis)


def _tc_post_chunk(bpair, hp_k, dis, b_k, n):
    nbk = n // NT_ROWS

    def body(p_ref, hp_ref, d_ref, b_ref, z_ref, s_ref):
        i = pl.program_id(0)
        agg = p_ref[1] - p_ref[0]
        z = jnp.maximum((agg + hp_ref[...]) * d_ref[...] + b_ref[...], 0.0)
        z_ref[...] = z
        cs = jnp.sum(z, axis=0)
        cs2 = jnp.sum(z * z, axis=0)

        @pl.when(i == 0)
        def _():
            s_ref[0, :] = cs
            s_ref[1, :] = cs2

        @pl.when(i > 0)
        def _():
            s_ref[0, :] += cs
            s_ref[1, :] += cs2

    return pl.pallas_call(
        body,
        grid=(nbk,),
        in_specs=[
            pl.BlockSpec((2, NT_ROWS, 128), lambda i: (0, i, 0)),
            pl.BlockSpec((NT_ROWS, 128), lambda i: (i, 0)),
            pl.BlockSpec((NT_ROWS, 1), lambda i: (i, 0)),
            pl.BlockSpec((1, 128), lambda i: (0, 0)),
        ],
        out_specs=[
            pl.BlockSpec((NT_ROWS, 128), lambda i: (i, 0)),
            pl.BlockSpec((2, 128), lambda i: (0, 0)),
        ],
        out_shape=[
            jax.ShapeDtypeStruct((n, 128), jnp.float32),
            jax.ShapeDtypeStruct((2, 128), jnp.float32),
        ],
    )(bpair, hp_k, dis, b_k)


def _tc_pool_fc(z_ch, sums, gam, bet, batch2d, fcw, fcb2d, n):
    nbk = n // NT_ROWS
    ncls = fcw.shape[1]

    def body(s_ref, g_ref, be_ref, z_ref, bat_ref, fw_ref, fb_ref, o_ref, emb):
        i = pl.program_id(0)
        s0 = s_ref[0, 0, :]
        s1 = s_ref[0, 1, :]
        mu = s0 / n
        var = s1 / n - mu * mu
        scale = g_ref[0, 0, :] * lax.rsqrt(var + EPS)
        shift = be_ref[0, 0, :] - mu * scale
        a = z_ref[0] * scale[None, :] + shift[None, :]
        bb = bat_ref[...]

        @pl.when(i == 0)
        def _():
            emb[...] = jnp.full((NGRAPH, 128), -jnp.inf, jnp.float32)

        for g in range(NGRAPH):
            vals = jnp.where(bb == g, a, -jnp.inf)
            emb[g, :] = jnp.maximum(emb[g, :], jnp.max(vals, axis=0))

        @pl.when(i == nbk - 1)
        def _():
            o_ref[...] = lax.dot_general(
                emb[...], fw_ref[...], (((1,), (0,)), ((), ())),
                preferred_element_type=jnp.float32) + fb_ref[...]

    return pl.pallas_call(
        body,
        grid=(nbk,),
        in_specs=[
            pl.BlockSpec((1, 2, 128), lambda i: (0, 0, 0)),
            pl.BlockSpec((1, 1, 128), lambda i: (0, 0, 0)),
            pl.BlockSpec((1, 1, 128), lambda i: (0, 0, 0)),
            pl.BlockSpec((1, NT_ROWS, 128), lambda i: (0, i, 0)),
            pl.BlockSpec((NT_ROWS, 1), lambda i: (i, 0)),
            pl.BlockSpec((128, ncls), lambda i: (0, 0)),
            pl.BlockSpec((1, ncls), lambda i: (0, 0)),
        ],
        out_specs=pl.BlockSpec((NGRAPH, ncls), lambda i: (0, 0)),
        out_shape=jax.ShapeDtypeStruct((NGRAPH, ncls), jnp.float32),
        scratch_shapes=[pltpu.VMEM((NGRAPH, 128), jnp.float32)],
    )(sums, gam, bet, z_ch, batch2d, fcw, fcb2d)


def kernel(x, edge_index, batch, Ws, bs, gammas, betas, fcW, fcb):
    n = x.shape[0]
    e = edge_index.shape[1]
    nb = -(-e // (NW * EB))
    e_pad = NW * nb * EB
    e_pad2 = -(-(e_pad + NW) // CB) * CB + CB
    zrow = e_pad2
    nc_max = max(_padc(w.shape[1]) for w in Ws) // 128
    nbb = -(-(n + 1) // (NW * EB))

    order = jnp.argsort(edge_index[1])
    ss = edge_index[0][order]
    offs = jnp.searchsorted(
        edge_index[1][order], jnp.arange(n + 1, dtype=jnp.int32)
    ).astype(jnp.int32)
    src_pad = jnp.concatenate([ss, jnp.zeros((e_pad - e,), jnp.int32)])
    offc = (jnp.arange(nc_max, dtype=jnp.int32) * n)[:, None]
    srcb_off = (src_pad[None, :] + offc).reshape(nc_max, NW, nb, EB)

    def bound_idx(o):
        g = jnp.where(o > 0, o - 1, zrow)
        return _pad_to(g, (NW * nbb * EB,)).at[n + 1:].set(zrow)

    bidx = jnp.stack([bound_idx(offs[:n + 1]),
                      bound_idx(jnp.concatenate(
                          [offs[1:], jnp.full((1,), e, jnp.int32)]))])
    bidx = bidx.reshape(2, NW, nbb, EB)

    dis = _tc_dis(offs[:n].reshape(n, 1), offs[1:].reshape(n, 1), n)

    c0 = x.shape[1]
    a_ch = x.reshape(1, n, c0)
    nc_in = 1
    sums = jnp.stack([jnp.zeros((c0,), jnp.float32),
                      jnp.full((c0,), n * (1.0 - EPS), jnp.float32)])
    sums = sums.reshape(1, 2, c0)
    gam = jnp.ones((1, 1, c0), jnp.float32)
    bet = jnp.zeros((1, 1, c0), jnp.float32)

    for li, (wmat, b) in enumerate(zip(Ws, bs)):
        pc_in = nc_in * 128
        pc_out = _padc(wmat.shape[1])
        nc = pc_out // 128
        wp = _pad_to(wmat, (pc_in, pc_out))
        hp = _tc_matmul(a_ch, wp, sums, gam, bet, dis, n, nc_in, nc)
        hp_flat = hp.reshape(nc * n, 128)
        b_pad = _pad_to(b, (pc_out,)).reshape(nc, 1, 128)
        z_list, s_list = [], []
        for kc in range(nc):
            msg = _sc_gather_msg(hp_flat, srcb_off[kc], e_pad2, nb)
            p_tab = _tc_cumsum(msg, e_pad2)
            bpair = _sc_gather_bounds(p_tab, bidx, nbb)
            z_k, s_k = _tc_post_chunk(bpair, hp[kc], dis, b_pad[kc], n)
            z_list.append(z_k)
            s_list.append(s_k)
        a_ch = jnp.stack(z_list)
        sums = jnp.stack(s_list)
        gam = _pad_to(gammas[li], (pc_out,)).reshape(nc, 1, 128)
        bet = _pad_to(betas[li], (pc_out,)).reshape(nc, 1, 128)
        nc_in = nc

    return _tc_pool_fc(a_ch, sums, gam, bet,
                       batch.reshape(n, 1), fcW,
                       fcb.reshape(1, -1), n)


# speedup vs baseline: 1.6355x; 1.0342x over previous
import functools

import jax
import jax.numpy as jnp
from jax import lax
from jax.experimental import pallas as pl
from jax.experimental.pallas import tpu as pltpu
from jax.experimental.pallas import tpu_sc as plsc

NSC = 2
NT = 16
NW = NSC * NT
EB = 128
NGRAPH = 16
EPS = 1e-5
NT_ROWS = 1000
CB = 1024


def _padc(c):
    return ((c + 127) // 128) * 128


def _pad_to(a, shape):
    return jnp.pad(a, [(0, t - s) for s, t in zip(a.shape, shape)])


def _sc_gather_msg(hp_flat, srcb_k, e_pad2, nb):
    mesh = plsc.VectorSubcoreMesh(core_axis_name="c", subcore_axis_name="s")
    per_w = nb * EB
    nz = (e_pad2 - NW * per_w - CB) // NW
    e_pad = NW * per_w

    nq = 4
    nfull = nb // nq

    @functools.partial(
        pl.kernel, mesh=mesh,
        out_type=jax.ShapeDtypeStruct((e_pad2, 128), jnp.float32),
        scratch_types=[
            pltpu.VMEM((nb, EB), jnp.int32),
            [pltpu.VMEM((EB, 128), jnp.float32) for _ in range(nq)],
            pltpu.VMEM((max(nz, 8), 128), jnp.float32),
            [pltpu.SemaphoreType.DMA for _ in range(nq)],
        ],
    )
    def k(hp_hbm, src_hbm, out_hbm, idx, bufs, zbuf, sems):
        c = lax.axis_index("c")
        s = lax.axis_index("s")
        w = s * NSC + c

        def zb(t, _):
            r = t // 8
            col = (t % 8) * 16
            zbuf[r, pl.ds(col, 16)] = jnp.zeros((16,), jnp.float32)
            return 0

        lax.fori_loop(0, max(nz, 8) * 8, zb, 0)
        pltpu.sync_copy(src_hbm.at[w], idx)

        def quad(t, _):
            j0 = t * nq
            cps = [pltpu.async_copy(hp_hbm.at[idx.at[j0 + q]], bufs[q],
                                    sems[q]) for q in range(nq)]
            for q in range(nq):
                cps[q].wait()
                pltpu.sync_copy(
                    bufs[q],
                    out_hbm.at[pl.ds(w * per_w + (j0 + q) * EB, EB)])
            return 0

        lax.fori_loop(0, nfull, quad, 0)
        for j in range(nfull * nq, nb):
            pltpu.async_copy(hp_hbm.at[idx.at[j]], bufs[0], sems[0]).wait()
            pltpu.sync_copy(bufs[0],
                            out_hbm.at[pl.ds(w * per_w + j * EB, EB)])
        if nz > 0:
            pltpu.sync_copy(zbuf.at[pl.ds(0, nz)],
                            out_hbm.at[pl.ds(e_pad + w * nz, nz)])

    return k(hp_flat, srcb_k)


def _sc_gather_bounds(p_tab, bidx, nbb):
    mesh = plsc.VectorSubcoreMesh(core_axis_name="c", subcore_axis_name="s")
    per_w = nbb * EB

    @functools.partial(
        pl.kernel, mesh=mesh,
        out_type=jax.ShapeDtypeStruct((2, NW * per_w, 128), jnp.float32),
        scratch_types=[
            pltpu.VMEM((nbb, EB), jnp.int32),
            pltpu.VMEM((EB, 128), jnp.float32),
            pltpu.SemaphoreType.DMA,
        ],
    )
    def k(p_hbm, bidx_hbm, out_hbm, idx, buf, sem):
        c = lax.axis_index("c")
        s = lax.axis_index("s")
        w = s * NSC + c
        for h in range(2):
            pltpu.sync_copy(bidx_hbm.at[h, w], idx)
            for j in range(nbb):
                pltpu.async_copy(p_hbm.at[idx.at[j]], buf, sem).wait()
                pltpu.sync_copy(
                    buf, out_hbm.at[h, pl.ds(w * per_w + j * EB, EB)])

    return k(p_tab, bidx)


def _tc_cumsum(msg, e_pad2):
    nblk = e_pad2 // CB

    def body(m_ref, p_ref, carry):
        i = pl.program_id(0)

        @pl.when(i == 0)
        def _():
            carry[...] = jnp.zeros((1, 128), jnp.float32)

        @pl.when(i < nblk)
        def _():
            x = m_ref[...]
            sh = 1
            while sh < CB:
                x = x + jnp.concatenate(
                    [jnp.zeros((sh, 128), jnp.float32), x[:CB - sh]], axis=0)
                sh *= 2
            y = x + carry[...]
            p_ref[...] = y
            carry[...] = y[CB - 1:CB, :]

        @pl.when(i == nblk)
        def _():
            p_ref[...] = jnp.zeros((CB, 128), jnp.float32)

    return pl.pallas_call(
        body,
        grid=(nblk + 1,),
        in_specs=[pl.BlockSpec((CB, 128),
                               lambda i: (jnp.minimum(i, nblk - 1), 0))],
        out_specs=pl.BlockSpec((CB, 128), lambda i: (i, 0)),
        out_shape=jax.ShapeDtypeStruct((e_pad2 + CB, 128), jnp.float32),
        scratch_shapes=[pltpu.VMEM((1, 128), jnp.float32)],
    )(msg)


def _tc_dis(olo, ohi, n):
    nbk = n // NT_ROWS

    def body(lo_ref, hi_ref, o_ref):
        deg = (hi_ref[...] - lo_ref[...]).astype(jnp.float32) + 1.0
        o_ref[...] = lax.rsqrt(deg)

    return pl.pallas_call(
        body,
        grid=(nbk,),
        in_specs=[pl.BlockSpec((NT_ROWS, 1), lambda i: (i, 0)),
                  pl.BlockSpec((NT_ROWS, 1), lambda i: (i, 0))],
        out_specs=pl.BlockSpec((NT_ROWS, 1), lambda i: (i, 0)),
        out_shape=jax.ShapeDtypeStruct((n, 1), jnp.float32),
    )(olo, ohi)


def _tc_matmul(a_ch, wmat, sums_in, gam_in, bet_in, dis, n, nc_in, nc_out):
    nbk = n // NT_ROWS
    c_in = nc_in * 128

    def body(a_ref, w_ref, s_ref, g_ref, be_ref, d_ref, o_ref):
        acc = None
        for ki in range(nc_in):
            s0 = s_ref[ki, 0, :]
            s1 = s_ref[ki, 1, :]
            mu = s0 / n
            var = s1 / n - mu * mu
            scale = g_ref[ki, 0, :] * lax.rsqrt(var + EPS)
            shift = be_ref[ki, 0, :] - mu * scale
            aeff = a_ref[ki] * scale[None, :] + shift[None, :]
            part = lax.dot_general(
                aeff, w_ref[ki * 128:(ki + 1) * 128, :],
                (((1,), (0,)), ((), ())), preferred_element_type=jnp.float32)
            acc = part if acc is None else acc + part
        o_ref[0] = acc * d_ref[...]

    return pl.pallas_call(
        body,
        grid=(nbk, nc_out),
        in_specs=[
            pl.BlockSpec((nc_in, NT_ROWS, 128), lambda i, k: (0, i, 0)),
            pl.BlockSpec((c_in, 128), lambda i, k: (0, k)),
            pl.BlockSpec((nc_in, 2, 128), lambda i, k: (0, 0, 0)),
            pl.BlockSpec((nc_in, 1, 128), lambda i, k: (0, 0, 0)),
            pl.BlockSpec((nc_in, 1, 128), lambda i, k: (0, 0, 0)),
            pl.BlockSpec((NT_ROWS, 1), lambda i, k: (i, 0)),
        ],
        out_specs=pl.BlockSpec((1, NT_ROWS, 128), lambda i, k: (k, i, 0)),
        out_shape=jax.ShapeDtypeStruct((nc_out, n, 128), jnp.float32),
    )(a_ch, wmat, sums_in, gam_in, bet_in, dis)


def _tc_post_chunk(bpair, hp_k, dis, b_k, n):
    nbk = n // NT_ROWS

    def body(p_ref, hp_ref, d_ref, b_ref, z_ref, s_ref):
        i = pl.program_id(0)
        agg = p_ref[1] - p_ref[0]
        z = jnp.maximum((agg + hp_ref[...]) * d_ref[...] + b_ref[...], 0.0)
        z_ref[...] = z
        cs = jnp.sum(z, axis=0)
        cs2 = jnp.sum(z * z, axis=0)

        @pl.when(i == 0)
        def _():
            s_ref[0, :] = cs
            s_ref[1, :] = cs2

        @pl.when(i > 0)
        def _():
            s_ref[0, :] += cs
            s_ref[1, :] += cs2

    return pl.pallas_call(
        body,
        grid=(nbk,),
        in_specs=[
            pl.BlockSpec((2, NT_ROWS, 128), lambda i: (0, i, 0)),
            pl.BlockSpec((NT_ROWS, 128), lambda i: (i, 0)),
            pl.BlockSpec((NT_ROWS, 1), lambda i: (i, 0)),
            pl.BlockSpec((1, 128), lambda i: (0, 0)),
        ],
        out_specs=[
            pl.BlockSpec((NT_ROWS, 128), lambda i: (i, 0)),
            pl.BlockSpec((2, 128), lambda i: (0, 0)),
        ],
        out_shape=[
            jax.ShapeDtypeStruct((n, 128), jnp.float32),
            jax.ShapeDtypeStruct((2, 128), jnp.float32),
        ],
    )(bpair, hp_k, dis, b_k)


def _tc_pool_fc(z_ch, sums, gam, bet, batch2d, fcw, fcb2d, n):
    nbk = n // NT_ROWS
    ncls = fcw.shape[1]

    def body(s_ref, g_ref, be_ref, z_ref, bat_ref, fw_ref, fb_ref, o_ref, emb):
        i = pl.program_id(0)
        s0 = s_ref[0, 0, :]
        s1 = s_ref[0, 1, :]
        mu = s0 / n
        var = s1 / n - mu * mu
        scale = g_ref[0, 0, :] * lax.rsqrt(var + EPS)
        shift = be_ref[0, 0, :] - mu * scale
        a = z_ref[0] * scale[None, :] + shift[None, :]
        bb = bat_ref[...]

        @pl.when(i == 0)
        def _():
            emb[...] = jnp.full((NGRAPH, 128), -jnp.inf, jnp.float32)

        for g in range(NGRAPH):
            vals = jnp.where(bb == g, a, -jnp.inf)
            emb[g, :] = jnp.maximum(emb[g, :], jnp.max(vals, axis=0))

        @pl.when(i == nbk - 1)
        def _():
            o_ref[...] = lax.dot_general(
                emb[...], fw_ref[...], (((1,), (0,)), ((), ())),
                preferred_element_type=jnp.float32) + fb_ref[...]

    return pl.pallas_call(
        body,
        grid=(nbk,),
        in_specs=[
            pl.BlockSpec((1, 2, 128), lambda i: (0, 0, 0)),
            pl.BlockSpec((1, 1, 128), lambda i: (0, 0, 0)),
            pl.BlockSpec((1, 1, 128), lambda i: (0, 0, 0)),
            pl.BlockSpec((1, NT_ROWS, 128), lambda i: (0, i, 0)),
            pl.BlockSpec((NT_ROWS, 1), lambda i: (i, 0)),
            pl.BlockSpec((128, ncls), lambda i: (0, 0)),
            pl.BlockSpec((1, ncls), lambda i: (0, 0)),
        ],
        out_specs=pl.BlockSpec((NGRAPH, ncls), lambda i: (0, 0)),
        out_shape=jax.ShapeDtypeStruct((NGRAPH, ncls), jnp.float32),
        scratch_shapes=[pltpu.VMEM((NGRAPH, 128), jnp.float32)],
    )(sums, gam, bet, z_ch, batch2d, fcw, fcb2d)


def kernel(x, edge_index, batch, Ws, bs, gammas, betas, fcW, fcb):
    n = x.shape[0]
    e = edge_index.shape[1]
    nb = -(-e // (NW * EB))
    e_pad = NW * nb * EB
    e_pad2 = -(-(e_pad + NW) // CB) * CB + CB
    zrow = e_pad2
    nc_max = max(_padc(w.shape[1]) for w in Ws) // 128
    nbb = -(-(n + 1) // (NW * EB))

    order = jnp.argsort(edge_index[1])
    ss = edge_index[0][order]
    offs = jnp.searchsorted(
        edge_index[1][order], jnp.arange(n + 1, dtype=jnp.int32)
    ).astype(jnp.int32)
    src_pad = jnp.concatenate([ss, jnp.zeros((e_pad - e,), jnp.int32)])
    offc = (jnp.arange(nc_max, dtype=jnp.int32) * n)[:, None]
    srcb_off = (src_pad[None, :] + offc).reshape(nc_max, NW, nb, EB)

    def bound_idx(o):
        g = jnp.where(o > 0, o - 1, zrow)
        return _pad_to(g, (NW * nbb * EB,)).at[n + 1:].set(zrow)

    bidx = jnp.stack([bound_idx(offs[:n + 1]),
                      bound_idx(jnp.concatenate(
                          [offs[1:], jnp.full((1,), e, jnp.int32)]))])
    bidx = bidx.reshape(2, NW, nbb, EB)

    dis = _tc_dis(offs[:n].reshape(n, 1), offs[1:].reshape(n, 1), n)

    c0 = x.shape[1]
    a_ch = x.reshape(1, n, c0)
    nc_in = 1
    sums = jnp.stack([jnp.zeros((c0,), jnp.float32),
                      jnp.full((c0,), n * (1.0 - EPS), jnp.float32)])
    sums = sums.reshape(1, 2, c0)
    gam = jnp.ones((1, 1, c0), jnp.float32)
    bet = jnp.zeros((1, 1, c0), jnp.float32)

    for li, (wmat, b) in enumerate(zip(Ws, bs)):
        pc_in = nc_in * 128
        pc_out = _padc(wmat.shape[1])
        nc = pc_out // 128
        wp = _pad_to(wmat, (pc_in, pc_out))
        hp = _tc_matmul(a_ch, wp, sums, gam, bet, dis, n, nc_in, nc)
        hp_flat = hp.reshape(nc * n, 128)
        b_pad = _pad_to(b, (pc_out,)).reshape(nc, 1, 128)
        z_list, s_list = [], []
        for kc in range(nc):
            msg = _sc_gather_msg(hp_flat, srcb_off[kc], e_pad2, nb)
            p_tab = _tc_cumsum(msg, e_pad2)
            bpair = _sc_gather_bounds(p_tab, bidx, nbb)
            z_k, s_k = _tc_post_chunk(bpair, hp[kc], dis, b_pad[kc], n)
            z_list.append(z_k)
            s_list.append(s_k)
        a_ch = jnp.stack(z_list)
        sums = jnp.stack(s_list)
        gam = _pad_to(gammas[li], (pc_out,)).reshape(nc, 1, 128)
        bet = _pad_to(betas[li], (pc_out,)).reshape(nc, 1, 128)
        nc_in = nc

    return _tc_pool_fc(a_ch, sums, gam, bet,
                       batch.reshape(n, 1), fcW,
                       fcb.reshape(1, -1), n)
